# Initial kernel scaffold; baseline (speedup 1.0000x reference)
import jax
import jax.numpy as jnp
from jax.experimental import pallas as pl


def kernel(rpn_cls_scores, rpn_bbox_preds, img_size):
    raise NotImplementedError("write your pallas kernel here")


# speedup vs baseline: 21.6834x; 21.6834x over previous
import numpy as np
import jax
import jax.numpy as jnp
from jax.experimental import pallas as pl
from jax.experimental.pallas import tpu as pltpu

_FEAT_STRIDE = 16
_N_ANCHOR = 9
_NMS_THRESH = 0.7
_PRE_NMS = 12000
_POST_NMS = 2000
_MIN_SIZE = 16.0
_FH = 64
_FW = 64
_N = _N_ANCHOR * _FH * _FW
_ROWS = 288
_NEG_INF = float("-inf")


def _base_anchors(base_size=16, ratios=(0.5, 1.0, 2.0), scales=(8, 16, 32)):
    ab = np.zeros((len(ratios) * len(scales), 4), dtype=np.float32)
    px = base_size / 2.0
    py = base_size / 2.0
    for i, r in enumerate(ratios):
        for j, s in enumerate(scales):
            h = base_size * s * np.sqrt(r)
            w = base_size * s * np.sqrt(1.0 / r)
            k = i * len(scales) + j
            ab[k, 0] = px - w / 2.0
            ab[k, 1] = py - h / 2.0
            ab[k, 2] = px + w / 2.0
            ab[k, 3] = py + h / 2.0
    return ab


def _plane_constants():
    base = _base_anchors()
    shift_x = np.arange(_FW, dtype=np.float32) * _FEAT_STRIDE
    shift_y = np.arange(_FH, dtype=np.float32) * _FEAT_STRIDE
    sx, sy = np.meshgrid(shift_x, shift_y)
    shifts = np.stack([sx.ravel(), sy.ravel(), sx.ravel(), sy.ravel()], axis=1)
    anchors = (shifts[:, None, :] + base[None, :, :]).reshape(-1, 4)
    anchors = anchors.astype(np.float32)
    widths = anchors[:, 2] - anchors[:, 0] + np.float32(1.0)
    heights = anchors[:, 3] - anchors[:, 1] + np.float32(1.0)
    ctr_x = anchors[:, 0] + np.float32(0.5) * widths
    ctr_y = anchors[:, 1] + np.float32(0.5) * heights
    flat = np.arange(_N, dtype=np.int32)

    def to_plane(v):
        return np.ascontiguousarray(v.reshape(4096, 9).T).reshape(_ROWS, 128)

    return (
        jnp.asarray(to_plane(widths)),
        jnp.asarray(to_plane(heights)),
        jnp.asarray(to_plane(ctr_x)),
        jnp.asarray(to_plane(ctr_y)),
        jnp.asarray(to_plane(flat)),
    )


_AW, _AH, _ACX, _ACY, _FLAT = _plane_constants()


def _proposal_kernel(c0, c1, dxr, dyr, dwr, dhr, aw, ah, acx, acy, flat_r,
                     img, out_ref):
    imh = img[0, 0]
    imw = img[0, 1]

    a0 = c0[...]
    a1 = c1[...]
    mx = jnp.maximum(a0, a1)
    e0 = jnp.exp(a0 - mx)
    e1 = jnp.exp(a1 - mx)
    score = e1 / (e0 + e1)

    widths = aw[...]
    heights = ah[...]
    pcx = dxr[...] * widths + acx[...]
    pcy = dyr[...] * heights + acy[...]
    pw = jnp.exp(dwr[...]) * widths
    ph = jnp.exp(dhr[...]) * heights
    x1 = jnp.clip(pcx - 0.5 * pw, 0.0, imw - 1.0)
    y1 = jnp.clip(pcy - 0.5 * ph, 0.0, imh - 1.0)
    x2 = jnp.clip(pcx + 0.5 * pw, 0.0, imw - 1.0)
    y2 = jnp.clip(pcy + 0.5 * ph, 0.0, imh - 1.0)
    ws = x2 - x1 + 1.0
    hs = y2 - y1 + 1.0
    valid = (ws >= _MIN_SIZE) & (hs >= _MIN_SIZE)
    neg = jnp.float32(_NEG_INF)
    s = jnp.where(valid, score, neg)
    flat = flat_r[...]

    u = jax.lax.bitcast_convert_type(s, jnp.int32)
    k = jnp.int32(_PRE_NMS)

    def cnt_ge(t):
        return jnp.sum(jnp.where(u >= t, jnp.int32(1), jnp.int32(0)))

    def bisect_val(i, lohi):
        lo, hi = lohi
        mid = lo + (hi - lo) // 2
        ge = cnt_ge(mid) >= k
        return (jnp.where(ge, mid, lo), jnp.where(ge, hi, mid))

    lo0 = jnp.int32(np.float32(_NEG_INF).view(np.int32))
    hi0 = jnp.int32(np.float32(2.0).view(np.int32))
    tstar, _ = jax.lax.fori_loop(0, 32, bisect_val, (lo0, hi0))

    cnt_gt = jnp.sum(jnp.where(u > tstar, jnp.int32(1), jnp.int32(0)))
    need = k - cnt_gt
    tie = u == tstar

    def cnt_tie_le(j):
        return jnp.sum(jnp.where(tie & (flat <= j), jnp.int32(1),
                                 jnp.int32(0)))

    def bisect_idx(i, lohi):
        lo, hi = lohi
        mid = lo + (hi - lo) // 2
        ge = cnt_tie_le(mid) >= need
        return (jnp.where(ge, lo, mid), jnp.where(ge, mid, hi))

    _, jstar = jax.lax.fori_loop(0, 17, bisect_idx,
                                 (jnp.int32(-1), jnp.int32(_N - 1)))
    cand = (u > tstar) | (tie & (flat <= jstar))
    s = jnp.where(cand, s, neg)

    area = (x2 - x1 + 1.0) * (y2 - y1 + 1.0)
    out_ref[...] = jnp.zeros((2048, 8), jnp.float32)
    lane = jax.lax.broadcasted_iota(jnp.int32, (1, 8), 1)
    big = jnp.int32(2 ** 30)

    def cond(state):
        slot, sc, m = state
        return (slot < _POST_NMS) & (m > neg)

    def body(state):
        slot, sc, m = state
        sel = sc == m
        chosen = jnp.min(jnp.where(sel, flat, big))
        selm = flat == chosen
        bx1 = jnp.max(jnp.where(selm, x1, neg))
        by1 = jnp.max(jnp.where(selm, y1, neg))
        bx2 = jnp.max(jnp.where(selm, x2, neg))
        by2 = jnp.max(jnp.where(selm, y2, neg))
        barea = (bx2 - bx1 + 1.0) * (by2 - by1 + 1.0)
        xx1 = jnp.maximum(bx1, x1)
        yy1 = jnp.maximum(by1, y1)
        xx2 = jnp.minimum(bx2, x2)
        yy2 = jnp.minimum(by2, y2)
        iw = jnp.maximum(0.0, xx2 - xx1 + 1.0)
        ih = jnp.maximum(0.0, yy2 - yy1 + 1.0)
        inter = iw * ih
        iou = inter / (barea + area - inter)
        sc2 = jnp.where((iou > _NMS_THRESH) | selm, neg, sc)
        row = (jnp.where(lane == 1, bx1, 0.0) +
               jnp.where(lane == 2, by1, 0.0) +
               jnp.where(lane == 3, bx2, 0.0) +
               jnp.where(lane == 4, by2, 0.0))
        out_ref[pl.ds(slot, 1), :] = row
        return (slot + 1, sc2, jnp.max(sc2))

    jax.lax.while_loop(cond, body, (jnp.int32(0), s, jnp.max(s)))


def kernel(rpn_cls_scores, rpn_bbox_preds, img_size):
    cls = rpn_cls_scores[0].reshape(_N_ANCHOR, 2, _FH * _FW)
    c0 = cls[:, 0, :].reshape(_ROWS, 128)
    c1 = cls[:, 1, :].reshape(_ROWS, 128)
    bb = rpn_bbox_preds[0].reshape(_N_ANCHOR, 4, _FH * _FW)
    dx = bb[:, 0, :].reshape(_ROWS, 128)
    dy = bb[:, 1, :].reshape(_ROWS, 128)
    dw = bb[:, 2, :].reshape(_ROWS, 128)
    dh = bb[:, 3, :].reshape(_ROWS, 128)
    img = img_size.astype(jnp.float32).reshape(1, 2)

    out = pl.pallas_call(
        _proposal_kernel,
        out_shape=jax.ShapeDtypeStruct((2048, 8), jnp.float32),
    )(c0, c1, dx, dy, dw, dh, _AW, _AH, _ACX, _ACY, _FLAT, img)
    return out[:_POST_NMS, :5]
